# baseline (speedup 1.0000x reference)
import functools

import jax
import jax.numpy as jnp
from jax.experimental import pallas as pl
from jax.experimental.pallas import tpu as pltpu


def _vb_kernel(Hf, Wf, bbox_ref, obj_ref, frame_ref, wobj_ref, bobj_ref,
               wkey_ref, bkey_ref, wval_ref, bval_ref, wctx_ref, bctx_ref,
               out_ref, att_acc_ref):
    b = pl.program_id(0)
    nb = pl.num_programs(0)
    f32 = jnp.float32
    bf16 = jnp.bfloat16
    N = bbox_ref.shape[0]
    C, P = frame_ref.shape[1], frame_ref.shape[2]

    bx = bbox_ref[...]
    x1 = jnp.minimum(bx[:, 0:1], bx[:, 2:3])
    x2 = jnp.maximum(bx[:, 0:1], bx[:, 2:3])
    y1 = jnp.minimum(bx[:, 1:2], bx[:, 3:4])
    y2 = jnp.maximum(bx[:, 1:2], bx[:, 3:4])
    pos = jax.lax.broadcasted_iota(jnp.int32, (N, P), 1)
    yc = ((pos // Wf).astype(f32) + 0.5) * (1.0 / Hf)
    xc = ((pos % Wf).astype(f32) + 0.5) * (1.0 / Wf)
    mask = ((yc >= y1) & (yc <= y2) & (xc >= x1) & (xc <= x2)).astype(f32)
    denom = jnp.maximum(jnp.sum(mask, axis=1, keepdims=True), 1.0)
    onehot = (obj_ref[...] == b).astype(f32)
    mb = (mask * onehot).astype(bf16)

    frame_b = frame_ref[0]

    pooled = jax.lax.dot_general(mb, frame_b, (((1,), (1,)), ((), ())),
                                 preferred_element_type=f32) / denom
    q = jnp.maximum(
        jnp.dot(pooled.astype(bf16), wobj_ref[...], preferred_element_type=f32)
        + bobj_ref[...], 0.0)
    keym = jnp.maximum(
        jax.lax.dot_general(frame_b, wkey_ref[...], (((0,), (0,)), ((), ())),
                            preferred_element_type=f32) + bkey_ref[...], 0.0)
    valm = jnp.maximum(
        jax.lax.dot_general(frame_b, wval_ref[...], (((0,), (0,)), ((), ())),
                            preferred_element_type=f32) + bval_ref[...], 0.0)
    scores = jax.lax.dot_general(q, keym, (((1,), (1,)), ((), ())),
                                 preferred_element_type=f32)
    m = jnp.max(scores, axis=1, keepdims=True)
    e = jnp.exp(scores - m)
    attn = (e / jnp.sum(e, axis=1, keepdims=True)) * onehot
    att = jnp.dot(attn, valm, preferred_element_type=f32)

    @pl.when(b == 0)
    def _():
        out_ref[:, :C] = pooled
        att_acc_ref[...] = att

    @pl.when(b != 0)
    def _():
        out_ref[:, :C] += pooled
        att_acc_ref[...] += att

    @pl.when(b == nb - 1)
    def _():
        ctx = jnp.maximum(
            jnp.dot(att_acc_ref[...].astype(bf16), wctx_ref[...],
                    preferred_element_type=f32) + bctx_ref[...], 0.0)
        out_ref[:, C:] = ctx


@jax.jit
def kernel(frame_deep_features, bboxes, obj_slicing, W_obj, b_obj, W_key,
           b_key, W_val, b_val, W_ctx, b_ctx):
    B, C, Hf, Wf = frame_deep_features.shape
    N = bboxes.shape[0]
    P = Hf * Wf
    Dq = W_obj.shape[1]
    Dc = W_ctx.shape[1]
    bf16 = jnp.bfloat16
    frame_flat = frame_deep_features.reshape(B, C, P).astype(bf16)
    obj2 = obj_slicing.reshape(N, 1)

    return pl.pallas_call(
        functools.partial(_vb_kernel, Hf, Wf),
        grid=(B,),
        in_specs=[
            pl.BlockSpec((N, 4), lambda b: (0, 0)),
            pl.BlockSpec((N, 1), lambda b: (0, 0)),
            pl.BlockSpec((1, C, P), lambda b: (b, 0, 0)),
            pl.BlockSpec((C, Dq), lambda b: (0, 0)),
            pl.BlockSpec((1, Dq), lambda b: (0, 0)),
            pl.BlockSpec((C, Dq), lambda b: (0, 0)),
            pl.BlockSpec((1, Dq), lambda b: (0, 0)),
            pl.BlockSpec((C, Dq), lambda b: (0, 0)),
            pl.BlockSpec((1, Dq), lambda b: (0, 0)),
            pl.BlockSpec((Dq, Dc), lambda b: (0, 0)),
            pl.BlockSpec((1, Dc), lambda b: (0, 0)),
        ],
        out_specs=pl.BlockSpec((N, C + Dc), lambda b: (0, 0)),
        out_shape=jax.ShapeDtypeStruct((N, C + Dc), jnp.float32),
        scratch_shapes=[pltpu.VMEM((N, Dq), jnp.float32)],
    )(bboxes, obj2, frame_flat, W_obj.astype(bf16), b_obj.reshape(1, Dq),
      W_key.astype(bf16), b_key.reshape(1, Dq), W_val.astype(bf16),
      b_val.reshape(1, Dq), W_ctx.astype(bf16), b_ctx.reshape(1, Dc))


# speedup vs baseline: 1.1847x; 1.1847x over previous
import functools

import jax
import jax.numpy as jnp
from jax.experimental import pallas as pl
from jax.experimental.pallas import tpu as pltpu


def _vb_kernel(Hf, Wf, bbox_ref, obj_ref, frame_ref, wobj_ref, bobj_ref,
               wkey_ref, bkey_ref, wval_ref, bval_ref, wctx_ref, bctx_ref,
               out_ref, att_acc_ref):
    b = pl.program_id(0)
    nb = pl.num_programs(0)
    f32 = jnp.float32
    bf16 = jnp.bfloat16
    N = bbox_ref.shape[0]
    C, P = frame_ref.shape[1], frame_ref.shape[2]

    bx = bbox_ref[...]
    x1 = jnp.minimum(bx[:, 0:1], bx[:, 2:3])
    x2 = jnp.maximum(bx[:, 0:1], bx[:, 2:3])
    y1 = jnp.minimum(bx[:, 1:2], bx[:, 3:4])
    y2 = jnp.maximum(bx[:, 1:2], bx[:, 3:4])
    pos = jax.lax.broadcasted_iota(jnp.int32, (N, P), 1)
    yc = ((pos // Wf).astype(f32) + 0.5) * (1.0 / Hf)
    xc = ((pos % Wf).astype(f32) + 0.5) * (1.0 / Wf)
    mask = ((yc >= y1) & (yc <= y2) & (xc >= x1) & (xc <= x2)).astype(f32)
    denom = jnp.maximum(jnp.sum(mask, axis=1, keepdims=True), 1.0)
    onehot = (obj_ref[...] == b).astype(f32)
    mb = (mask * onehot).astype(bf16)

    frame_b = frame_ref[0].astype(bf16)

    pooled = jax.lax.dot_general(mb, frame_b, (((1,), (1,)), ((), ())),
                                 preferred_element_type=f32) / denom
    q = jnp.maximum(
        jnp.dot(pooled.astype(bf16), wobj_ref[...].astype(bf16),
                preferred_element_type=f32)
        + bobj_ref[...], 0.0)
    keym = jnp.maximum(
        jax.lax.dot_general(frame_b, wkey_ref[...].astype(bf16),
                            (((0,), (0,)), ((), ())),
                            preferred_element_type=f32) + bkey_ref[...], 0.0)
    valm = jnp.maximum(
        jax.lax.dot_general(frame_b, wval_ref[...].astype(bf16),
                            (((0,), (0,)), ((), ())),
                            preferred_element_type=f32) + bval_ref[...], 0.0)
    scores = jax.lax.dot_general(q, keym, (((1,), (1,)), ((), ())),
                                 preferred_element_type=f32)
    m = jnp.max(scores, axis=1, keepdims=True)
    e = jnp.exp(scores - m)
    attn = (e / jnp.sum(e, axis=1, keepdims=True)) * onehot
    att = jnp.dot(attn, valm, preferred_element_type=f32)

    @pl.when(b == 0)
    def _():
        out_ref[:, :C] = pooled
        att_acc_ref[...] = att

    @pl.when(b != 0)
    def _():
        out_ref[:, :C] += pooled
        att_acc_ref[...] += att

    @pl.when(b == nb - 1)
    def _():
        ctx = jnp.maximum(
            jnp.dot(att_acc_ref[...].astype(bf16), wctx_ref[...].astype(bf16),
                    preferred_element_type=f32) + bctx_ref[...], 0.0)
        out_ref[:, C:] = ctx


@jax.jit
def kernel(frame_deep_features, bboxes, obj_slicing, W_obj, b_obj, W_key,
           b_key, W_val, b_val, W_ctx, b_ctx):
    B, C, Hf, Wf = frame_deep_features.shape
    N = bboxes.shape[0]
    P = Hf * Wf
    Dq = W_obj.shape[1]
    Dc = W_ctx.shape[1]
    frame_flat = frame_deep_features.reshape(B, C, P)
    obj2 = obj_slicing.reshape(N, 1)

    return pl.pallas_call(
        functools.partial(_vb_kernel, Hf, Wf),
        grid=(B,),
        in_specs=[
            pl.BlockSpec((N, 4), lambda b: (0, 0)),
            pl.BlockSpec((N, 1), lambda b: (0, 0)),
            pl.BlockSpec((1, C, P), lambda b: (b, 0, 0)),
            pl.BlockSpec((C, Dq), lambda b: (0, 0)),
            pl.BlockSpec((1, Dq), lambda b: (0, 0)),
            pl.BlockSpec((C, Dq), lambda b: (0, 0)),
            pl.BlockSpec((1, Dq), lambda b: (0, 0)),
            pl.BlockSpec((C, Dq), lambda b: (0, 0)),
            pl.BlockSpec((1, Dq), lambda b: (0, 0)),
            pl.BlockSpec((Dq, Dc), lambda b: (0, 0)),
            pl.BlockSpec((1, Dc), lambda b: (0, 0)),
        ],
        out_specs=pl.BlockSpec((N, C + Dc), lambda b: (0, 0)),
        out_shape=jax.ShapeDtypeStruct((N, C + Dc), jnp.float32),
        scratch_shapes=[pltpu.VMEM((N, Dq), jnp.float32)],
    )(bboxes, obj2, frame_flat, W_obj, b_obj.reshape(1, Dq),
      W_key, b_key.reshape(1, Dq), W_val,
      b_val.reshape(1, Dq), W_ctx, b_ctx.reshape(1, Dc))
